# baseline (speedup 1.0000x reference)
import functools

import jax
import jax.numpy as jnp
from jax import lax
from jax.experimental import pallas as pl
from jax.experimental.pallas import tpu as pltpu, tpu_sc as plsc

N_PLAYERS = 1000000
GENERIC_ID = N_PLAYERS + 1
D = 32
NSLOT = 10


def _gather_body(n_chunks, chunk, table_hbm, idx_hbm, out_hbm, idx_v, rows_v, sem):
    nc = plsc.get_sparse_core_info().num_cores
    wid = lax.axis_index("s") * nc + lax.axis_index("c")
    per_w = n_chunks * chunk

    def step(i, _):
        base = wid * per_w + i * chunk
        pltpu.sync_copy(idx_hbm.at[pl.ds(base, chunk)], idx_v)
        pltpu.async_copy(table_hbm.at[idx_v], rows_v, sem).wait()
        pltpu.sync_copy(rows_v, out_hbm.at[pl.ds(base, chunk)])
        return ()

    lax.fori_loop(0, n_chunks, step, ())


@functools.partial(jax.jit, static_argnums=(2,))
def _sc_gather(table, ids, n_rows):
    info = plsc.get_sparse_core_info()
    nw = info.num_cores * info.num_subcores
    per_w = n_rows // nw
    chunk = 1024
    n_chunks = per_w // chunk
    mesh = plsc.VectorSubcoreMesh(core_axis_name="c", subcore_axis_name="s")
    kern = pl.kernel(
        functools.partial(_gather_body, n_chunks, chunk),
        out_type=jax.ShapeDtypeStruct((n_rows, D), jnp.float32),
        mesh=mesh,
        scratch_types=[
            pltpu.VMEM((chunk,), jnp.int32),
            pltpu.VMEM((chunk, D), jnp.float32),
            pltpu.SemaphoreType.DMA,
        ],
        compiler_params=pltpu.CompilerParams(use_tc_tiling_on_sc=False),
    )
    return kern(table, ids)


def _dense_body(emb_ref, ids_ref, w3_ref, g_ref, b2_ref, out_ref):
    emb = emb_ref[...]
    ids = ids_ref[...]
    w3 = w3_ref[...]
    g = g_ref[...]
    mask = (ids == GENERIC_ID).astype(jnp.float32)
    term1 = jnp.sum(emb * w3[None, :, :], axis=(1, 2))
    s_b = jnp.sum(emb, axis=1)
    wm = jnp.einsum("bs,sd->bd", mask, w3,
                    preferred_element_type=jnp.float32)
    term2 = jnp.sum(wm * (0.1 * s_b - g), axis=1)
    out_ref[...] = (term1 + term2 + b2_ref[0])[:, None]


def kernel(x, table, W2, b2):
    B = x.shape[0]
    ids = x[:, :, 0]
    flat_ids = ids.reshape(B * NSLOT)
    emb = _sc_gather(table, flat_ids, B * NSLOT)
    emb3 = emb.reshape(B, NSLOT, D)
    w3 = W2.reshape(NSLOT, D)
    g = lax.dynamic_slice(table, (GENERIC_ID, 0), (1, D))

    bb = 2048
    grid = (B // bb,)
    out = pl.pallas_call(
        _dense_body,
        grid=grid,
        in_specs=[
            pl.BlockSpec((bb, NSLOT, D), lambda i: (i, 0, 0)),
            pl.BlockSpec((bb, NSLOT), lambda i: (i, 0)),
            pl.BlockSpec((NSLOT, D), lambda i: (0, 0)),
            pl.BlockSpec((1, D), lambda i: (0, 0)),
            pl.BlockSpec(memory_space=pltpu.SMEM),
        ],
        out_specs=pl.BlockSpec((bb, 1), lambda i: (i, 0)),
        out_shape=jax.ShapeDtypeStruct((B, 1), jnp.float32),
    )(emb3, ids, w3, g, b2)
    return out


# speedup vs baseline: 1.0476x; 1.0476x over previous
import functools

import jax
import jax.numpy as jnp
from jax import lax
from jax.experimental import pallas as pl
from jax.experimental.pallas import tpu as pltpu, tpu_sc as plsc

N_PLAYERS = 1000000
GENERIC_ID = N_PLAYERS + 1
D = 32
NSLOT = 10
B = 16384
CB = 128
RPC = CB * NSLOT
WPC = CB * NSLOT * 2


def _fused_body(x_hbm, tbl_hbm, wbig_hbm, gbig_hbm, b2big_hbm, out_hbm,
                xv0, xv1, idx0, idx1, rows0, rows1, yv, wbig_v, gbig_v, b2v,
                semx0, semx1, semg0, semg1):
    info = plsc.get_sparse_core_info()
    nc = info.num_cores
    nw = nc * info.num_subcores
    per_w = B // nw
    n_chunks = per_w // CB
    wid = lax.axis_index("s") * nc + lax.axis_index("c")

    pltpu.sync_copy(wbig_hbm, wbig_v)
    pltpu.sync_copy(gbig_hbm, gbig_v)
    pltpu.sync_copy(b2big_hbm, b2v)

    iota = lax.iota(jnp.int32, 16)
    iota2 = iota * 2
    iota10 = iota * 10
    x_base = wid * per_w * NSLOT * 2

    xvs = (xv0, xv1)
    idxs = (idx0, idx1)
    rowss = (rows0, rows1)
    semxs = (semx0, semx1)
    semgs = (semg0, semg1)

    def start_x(i):
        pltpu.async_copy(x_hbm.at[pl.ds(x_base + i * WPC, WPC)],
                         xvs[i % 2], semxs[i % 2])

    def wait_x(i):
        pltpu.make_async_copy(x_hbm.at[pl.ds(x_base + i * WPC, WPC)],
                              xvs[i % 2], semxs[i % 2]).wait()

    def extract(i):
        xv, idxv = xvs[i % 2], idxs[i % 2]

        def body(k, _):
            vals = plsc.load_gather(xv, [iota2 + k * 32])
            idxv[pl.ds(k * 16, 16)] = vals
            return 0

        lax.fori_loop(0, RPC // 16, body, 0)

    def start_g(i):
        pltpu.async_copy(tbl_hbm.at[idxs[i % 2]], rowss[i % 2], semgs[i % 2])

    def wait_g(i):
        pltpu.make_async_copy(tbl_hbm.at[idxs[i % 2]], rowss[i % 2],
                              semgs[i % 2]).wait()

    zero = jnp.zeros((16,), jnp.float32)

    def compute(i):
        rows, idxv = rowss[i % 2], idxs[i % 2]

        def group(gi, _):
            rvecs = [iota10 + (gi * 160 + s) for s in range(NSLOT)]
            ms = [jnp.where(plsc.load_gather(idxv, [rvecs[s]]) == GENERIC_ID,
                            jnp.float32(1.0), jnp.float32(0.0))
                  for s in range(NSLOT)]

            def dbody(d, acc):
                dvec = jnp.full((16,), d, jnp.int32)
                sd = zero
                wmd = zero
                for s in range(NSLOT):
                    e = plsc.load_gather(rows, [rvecs[s], dvec])
                    wv = wbig_v[pl.ds((s * D + d) * 16, 16)]
                    acc = acc + wv * e
                    sd = sd + e
                    wmd = wmd + wv * ms[s]
                gv = gbig_v[pl.ds(d * 16, 16)]
                return acc + wmd * (jnp.float32(0.1) * sd - gv)

            acc = lax.fori_loop(0, D, dbody, b2v[...])
            yv[pl.ds(gi * 16, 16)] = acc
            return 0

        lax.fori_loop(0, CB // 16, group, 0)
        pltpu.sync_copy(yv, out_hbm.at[pl.ds(wid * per_w + i * CB, CB)])

    start_x(0)
    start_x(1)
    wait_x(0)
    extract(0)
    start_g(0)
    for i in range(n_chunks):
        if i + 1 < n_chunks:
            wait_x(i + 1)
            extract(i + 1)
            if i + 2 < n_chunks:
                start_x(i + 2)
            start_g(i + 1)
        wait_g(i)
        compute(i)


@jax.jit
def _fused(x_flat, table, wbig, gbig, b2big):
    kern = pl.kernel(
        _fused_body,
        out_type=jax.ShapeDtypeStruct((B,), jnp.float32),
        mesh=plsc.VectorSubcoreMesh(core_axis_name="c", subcore_axis_name="s"),
        scratch_types=[
            pltpu.VMEM((WPC,), jnp.int32),
            pltpu.VMEM((WPC,), jnp.int32),
            pltpu.VMEM((RPC,), jnp.int32),
            pltpu.VMEM((RPC,), jnp.int32),
            pltpu.VMEM((RPC, D), jnp.float32),
            pltpu.VMEM((RPC, D), jnp.float32),
            pltpu.VMEM((CB,), jnp.float32),
            pltpu.VMEM((NSLOT * D * 16,), jnp.float32),
            pltpu.VMEM((D * 16,), jnp.float32),
            pltpu.VMEM((16,), jnp.float32),
            pltpu.SemaphoreType.DMA,
            pltpu.SemaphoreType.DMA,
            pltpu.SemaphoreType.DMA,
            pltpu.SemaphoreType.DMA,
        ],
        compiler_params=pltpu.CompilerParams(use_tc_tiling_on_sc=False,
                                             needs_layout_passes=False),
    )
    return kern(x_flat, table, wbig, gbig, b2big)


def kernel(x, table, W2, b2):
    x_flat = x.reshape(-1)
    wbig = jnp.tile(W2.reshape(NSLOT * D, 1), (1, 16)).reshape(-1)
    g = lax.dynamic_slice(table, (GENERIC_ID, 0), (1, D))
    gbig = jnp.tile(g.reshape(D, 1), (1, 16)).reshape(-1)
    b2big = jnp.broadcast_to(b2, (16,)).astype(jnp.float32)
    y = _fused(x_flat, table, wbig, gbig, b2big)
    return y.reshape(B, 1)
